# baseline (speedup 1.0000x reference)
import functools

import jax
import jax.numpy as jnp
from jax import lax
from jax.experimental import pallas as pl
from jax.experimental.pallas import tpu as pltpu
from jax.experimental.pallas import tpu_sc as plsc

B = 16384
D = 64
L = 16

_info = plsc.get_sparse_core_info()
NC, NS = _info.num_cores, _info.num_subcores
NW = NC * NS
BPW = B // NW
CH = 64
NCHUNK = BPW // CH


def _sc_body(idx_hbm, w_hbm, v_hbm, u_hbm, out_hbm,
             idx_v, q_v, wgA, vgA, ugA, wgB, vgB, ugB, out_v, semA, semB):
    wid = lax.axis_index("s") * NC + lax.axis_index("c")

    pltpu.sync_copy(idx_hbm.at[wid], idx_v)

    for t in range(3):
        for ci in range(NCHUNK):
            for g in range(CH // L):
                src = idx_v[pl.ds(t * BPW + ci * CH + g * L, L)]
                q_v[t, ci, pl.ds(g * L, L)] = src >> 1

    lane = jnp.arange(L, dtype=jnp.int32)
    tabs = (w_hbm, v_hbm, u_hbm)
    bufsA = (wgA, vgA, ugA)
    bufsB = (wgB, vgB, ugB)

    def fire(ci, bufs, sem):
        for t in range(3):
            pltpu.async_copy(tabs[t].at[q_v.at[t, ci]], bufs[t], sem)

    def drain(bufs, sem):
        for t in range(3):
            pltpu.make_async_copy(tabs[t].at[q_v.at[0, 0]], bufs[t],
                                  sem).wait()

    def compute(ci, bufs):
        wg, vg, ug = bufs
        for g in range(CH // L):
            ivecs = [idx_v[pl.ds(t * BPW + ci * CH + g * L, L)]
                     for t in range(3)]
            offs = [(iv & 1) << 6 for iv in ivecs]
            thetas = jnp.zeros((L,), jnp.float32)
            for r in range(L):
                j = g * L + r
                ow, ov, ou = offs[0][r], offs[1][r], offs[2][r]
                acc = jnp.zeros((L,), jnp.float32)
                for c in range(D // L):
                    acc = (acc
                           + wg[j, pl.ds(ow + c * L, L)]
                           * vg[j, pl.ds(ov + c * L, L)]
                           * ug[j, pl.ds(ou + c * L, L)])
                theta = jnp.sum(acc)
                thetas = thetas + jnp.where(lane == r, theta, 0.0)
            probs = 1.0 / (1.0 + jnp.exp(-thetas))
            out_v[pl.ds(ci * CH + g * L, L)] = probs

    fire(0, bufsA, semA)
    fire(1, bufsB, semB)

    def body(m, carry):
        c0 = 2 * m
        drain(bufsA, semA)
        compute(c0, bufsA)
        fire(c0 + 2, bufsA, semA)
        drain(bufsB, semB)
        compute(c0 + 1, bufsB)
        fire(c0 + 3, bufsB, semB)
        return carry

    lax.fori_loop(0, NCHUNK // 2 - 1, body, 0)

    drain(bufsA, semA)
    compute(NCHUNK - 2, bufsA)
    drain(bufsB, semB)
    compute(NCHUNK - 1, bufsB)

    pltpu.sync_copy(out_v, out_hbm.at[pl.ds(wid * BPW, BPW)])


@functools.partial(jax.jit, static_argnums=())
def kernel(indices, W, V, U):
    Wl = W.reshape(W.shape[0] // 2, 2 * D)
    Vl = V.reshape(V.shape[0] // 2, 2 * D)
    Ul = U.reshape(U.shape[0] // 2, 2 * D)
    idx = indices.astype(jnp.int32).T
    idx = idx.reshape(3, NW, BPW).transpose(1, 0, 2).reshape(NW, 3 * BPW)

    mesh = plsc.VectorSubcoreMesh(core_axis_name="c", subcore_axis_name="s")
    run = pl.kernel(
        _sc_body,
        mesh=mesh,
        out_type=jax.ShapeDtypeStruct((B,), jnp.float32),
        scratch_types=[
            pltpu.VMEM((3 * BPW,), jnp.int32),
            pltpu.VMEM((3, NCHUNK, CH), jnp.int32),
            pltpu.VMEM((CH, 2 * D), jnp.float32),
            pltpu.VMEM((CH, 2 * D), jnp.float32),
            pltpu.VMEM((CH, 2 * D), jnp.float32),
            pltpu.VMEM((CH, 2 * D), jnp.float32),
            pltpu.VMEM((CH, 2 * D), jnp.float32),
            pltpu.VMEM((CH, 2 * D), jnp.float32),
            pltpu.VMEM((BPW,), jnp.float32),
            pltpu.SemaphoreType.DMA,
            pltpu.SemaphoreType.DMA,
        ],
        compiler_params=pltpu.CompilerParams(needs_layout_passes=False),
    )
    return run(idx, Wl, Vl, Ul)
